# baseline (speedup 1.0000x reference)
import functools

import numpy as np
import jax
import jax.numpy as jnp
from jax import lax
from jax.experimental import pallas as pl
from jax.experimental.pallas import tpu as pltpu
from jax.experimental.pallas import tpu_sc as plsc

_VOCAB = 1000000
_D = 64
_SEQ = 200
_BATCH = 4096

_NC = 2
_NS = 16
_NW = _NC * _NS
_LANES = 128
_SGRP = _SEQ // 8


def _pos_encoding(length, depth):
    d = depth // 2
    positions = np.arange(length)[:, np.newaxis]
    depths = np.arange(d)[np.newaxis, :] / d
    rads = positions / 10000 ** depths
    pe = np.concatenate([np.sin(rads), np.cos(rads)], axis=-1)
    return jnp.asarray(pe, dtype=jnp.float32)


def _sc_embed(table, xt, pe):
    mesh = plsc.VectorSubcoreMesh(core_axis_name="c", subcore_axis_name="s")

    @functools.partial(
        pl.kernel,
        mesh=mesh,
        compiler_params=pltpu.CompilerParams(
            use_tc_tiling_on_sc=True, needs_layout_passes=False),
        out_type=jax.ShapeDtypeStruct((_SEQ, 8, _NW, 8, _LANES), jnp.float32),
        scratch_types=[
            pltpu.VMEM((8, _LANES), jnp.int32),
            pltpu.VMEM((_SEQ, _LANES), jnp.float32),
            pltpu.VMEM((4, _LANES, _LANES), jnp.float32),
            pltpu.VMEM((_D, _LANES), jnp.float32),
            pltpu.VMEM((_D, _LANES), jnp.float32),
            pltpu.SemaphoreType.DMA,
            pltpu.SemaphoreType.DMA,
            pltpu.SemaphoreType.DMA,
            pltpu.SemaphoreType.DMA,
            pltpu.SemaphoreType.DMA,
            pltpu.SemaphoreType.DMA,
        ],
    )
    def k(table_hbm, xt_hbm, pe_hbm, out_hbm, idx_v, pe_v, gbuf, tbuf0,
          tbuf1, gsem0, gsem1, gsem2, gsem3, osem0, osem1):
        tbufs = (tbuf0, tbuf1)
        gsems = (gsem0, gsem1, gsem2, gsem3)
        osems = (osem0, osem1)
        wid = lax.axis_index("s") * _NC + lax.axis_index("c")
        pltpu.sync_copy(xt_hbm.at[0, wid], idx_v)
        pltpu.sync_copy(pe_hbm, pe_v)

        iota = lax.iota(jnp.int32, 16)
        d_vecs = [dg * 16 + iota for dg in range(4)]

        def gather_copies(s, b):
            cps = []
            for j in range(8):
                idxv = idx_v[s % 8, pl.ds(j * 16, 16)] >> 1
                cps.append(pltpu.make_async_copy(
                    table_hbm.at[idxv], gbuf.at[b, pl.ds(j * 16, 16)],
                    gsems[b]))
            return cps

        def out_copy(s, tb, r):
            return pltpu.make_async_copy(
                tbufs[tb].at[pl.ds(r * 8, 8)],
                out_hbm.at[s, r, wid], osems[tb])

        for cp in gather_copies(0, 0):
            cp.start()
        for cp in gather_copies(1, 1):
            cp.start()

        def body(g, carry):
            for b in range(4):
                s = 4 * g + b
                tb = b % 2

                @pl.when(jnp.logical_and(s % 8 == 6, s < _SEQ - 2))
                def _():
                    pltpu.sync_copy(xt_hbm.at[(s + 2) // 8, wid], idx_v)

                @pl.when(s < _SEQ - 2)
                def _():
                    for cp in gather_copies(s + 2, (b + 2) % 4):
                        cp.start()

                for cp in gather_copies(s, b):
                    cp.wait()

                @pl.when(s >= 2)
                def _():
                    for r in range(8):
                        out_copy(s - 2, tb, r).wait()

                pe_vregs = [pe_v[s, pl.ds(dg * 16, 16)] for dg in range(4)]
                tbuf2d = tbufs[tb]

                @plsc.parallel_loop(0, 8, unroll=1)
                def _(eg):
                    pvec = idx_v[s % 8, pl.ds(eg * 16, 16)]
                    for l in range(16):
                        half = (pvec[l] & 1) * 64
                        e = eg * 16 + l
                        esplat = jnp.full((16,), e, jnp.int32)
                        for dg in range(4):
                            v = (gbuf[b, e, pl.ds(half + dg * 16, 16)]
                                 + pe_vregs[dg])
                            plsc.store_scatter(
                                tbuf2d, [d_vecs[dg], esplat], v)

                for r in range(8):
                    out_copy(s, tb, r).start()
            return carry

        lax.fori_loop(0, _SEQ // 4, body, 0)
        for r in range(8):
            out_copy(_SEQ - 2, 0, r).wait()
        for r in range(8):
            out_copy(_SEQ - 1, 1, r).wait()

    return k(table, xt, pe)


def kernel(x, table):
    xt = (x.astype(jnp.int32).T
          .reshape(_SGRP, 8, _NW, _LANES).transpose(0, 2, 1, 3))
    pe = jnp.pad(_pos_encoding(_SEQ, _D), ((0, 0), (0, _D)))
    out5d = _sc_embed(table.reshape(_VOCAB // 2, 2 * _D), xt, pe)
    out = out5d.transpose(2, 4, 0, 1, 3).reshape(_BATCH, _SEQ, _D)
    return out


# speedup vs baseline: 1.0780x; 1.0780x over previous
import functools

import numpy as np
import jax
import jax.numpy as jnp
from jax import lax
from jax.experimental import pallas as pl
from jax.experimental.pallas import tpu as pltpu
from jax.experimental.pallas import tpu_sc as plsc

_D = 64
_SEQ = 200
_BATCH = 4096

_NC = 2
_NS = 16
_NW = _NC * _NS
_LANES = 128
_SGRP = _SEQ // 8


def _pos_encoding(length, depth):
    d = depth // 2
    positions = np.arange(length)[:, np.newaxis]
    depths = np.arange(d)[np.newaxis, :] / d
    rads = positions / 10000 ** depths
    pe = np.concatenate([np.sin(rads), np.cos(rads)], axis=-1)
    return jnp.asarray(pe, dtype=jnp.float32)


def _sc_embed(table, xt, pe):
    mesh = plsc.VectorSubcoreMesh(core_axis_name="c", subcore_axis_name="s")

    @functools.partial(
        pl.kernel,
        mesh=mesh,
        compiler_params=pltpu.CompilerParams(
            use_tc_tiling_on_sc=False, needs_layout_passes=False),
        out_type=jax.ShapeDtypeStruct((_SEQ, 8, _NW, 1024), jnp.float32),
        scratch_types=[
            pltpu.VMEM((_SGRP, 8, _LANES), jnp.int32),
            pltpu.VMEM((_SEQ, _D), jnp.float32),
            pltpu.VMEM((4, _LANES, _D), jnp.float32),
            pltpu.VMEM((8192,), jnp.float32),
            pltpu.VMEM((8192,), jnp.float32),
            pltpu.SMEM((64,), jnp.int32),
            pltpu.SemaphoreType.DMA,
            pltpu.SemaphoreType.DMA,
            pltpu.SemaphoreType.DMA,
            pltpu.SemaphoreType.DMA,
            pltpu.SemaphoreType.DMA,
            pltpu.SemaphoreType.DMA,
        ],
    )
    def k(table_hbm, xt_hbm, pe_hbm, out_hbm, idx_v, pe_v, gbuf, tbuf0,
          tbuf1, sidx, gsem0, gsem1, gsem2, gsem3, osem0, osem1):
        tbufs = (tbuf0, tbuf1)
        gsems = (gsem0, gsem1, gsem2, gsem3)
        osems = (osem0, osem1)
        wid = lax.axis_index("s") * _NC + lax.axis_index("c")
        pltpu.sync_copy(xt_hbm.at[:, wid], idx_v)
        pltpu.sync_copy(pe_hbm, pe_v)

        iota = lax.iota(jnp.int32, 16)
        flatbase_vecs = []
        for dg in range(4):
            d_vec = dg * 16 + iota
            flatbase_vecs.append(((d_vec >> 3) << 10) + ((d_vec & 7) << 7))

        def gather_copies(s, b):
            cps = []
            for j in range(8):
                idxv = idx_v[s // 8, s % 8, pl.ds(j * 16, 16)]
                cps.append(pltpu.make_async_copy(
                    table_hbm.at[idxv], gbuf.at[b, pl.ds(j * 16, 16)],
                    gsems[b]))
            return cps

        def out_copy(s, tb, r):
            return pltpu.make_async_copy(
                tbufs[tb].at[pl.ds(r * 1024, 1024)],
                out_hbm.at[s, r, wid], osems[tb])

        for cp in gather_copies(0, 0):
            cp.start()
        for cp in gather_copies(1, 1):
            cp.start()

        def body(g, carry):
            for b in range(4):
                s = 4 * g + b
                tb = b % 2

                @pl.when(s < _SEQ - 2)
                def _():
                    for cp in gather_copies(s + 2, (b + 2) % 4):
                        cp.start()

                for cp in gather_copies(s, b):
                    cp.wait()

                @pl.when(s >= 2)
                def _():
                    for r in range(8):
                        out_copy(s - 2, tb, r).wait()

                pe_vregs = [pe_v[s, pl.ds(dg * 16, 16)] for dg in range(4)]
                tbuf2d = tbufs[tb]

                @plsc.parallel_loop(0, _LANES, unroll=4)
                def _(e):
                    esplat = jnp.full((16,), e, jnp.int32)
                    for dg in range(4):
                        v = gbuf[b, e, pl.ds(dg * 16, 16)] + pe_vregs[dg]
                        plsc.store_scatter(
                            tbuf2d, [flatbase_vecs[dg] + esplat], v)

                for r in range(8):
                    out_copy(s, tb, r).start()
            return carry

        lax.fori_loop(0, _SEQ // 4, body, 0)
        for r in range(8):
            out_copy(_SEQ - 2, 0, r).wait()
        for r in range(8):
            out_copy(_SEQ - 1, 1, r).wait()

    return k(table, xt, pe)


def kernel(x, table):
    xt = (x.astype(jnp.int32).T
          .reshape(_SGRP, 8, _NW, _LANES).transpose(0, 2, 1, 3))
    pe = _pos_encoding(_SEQ, _D)
    out4d = _sc_embed(table, xt, pe)
    out = (out4d.reshape(_SEQ, 8, _NW, 8, _LANES)
           .transpose(2, 4, 0, 1, 3).reshape(_BATCH, _SEQ, _D))
    return out


# speedup vs baseline: 1.8835x; 1.7472x over previous
import functools

import numpy as np
import jax
import jax.numpy as jnp
from jax import lax
from jax.experimental import pallas as pl
from jax.experimental.pallas import tpu as pltpu
from jax.experimental.pallas import tpu_sc as plsc

_D = 64
_SEQ = 200
_BATCH = 4096

_NC = 2
_NS = 16
_NW = _NC * _NS
_LANES = 128
_SGRP = _SEQ // 8


def _pos_encoding(length, depth):
    d = depth // 2
    positions = np.arange(length)[:, np.newaxis]
    depths = np.arange(d)[np.newaxis, :] / d
    rads = positions / 10000 ** depths
    pe = np.concatenate([np.sin(rads), np.cos(rads)], axis=-1)
    return jnp.asarray(pe, dtype=jnp.float32)


def _sc_embed(table, xt, pe):
    mesh = plsc.VectorSubcoreMesh(core_axis_name="c", subcore_axis_name="s")

    @functools.partial(
        pl.kernel,
        mesh=mesh,
        compiler_params=pltpu.CompilerParams(
            use_tc_tiling_on_sc=False, needs_layout_passes=False),
        out_type=jax.ShapeDtypeStruct((_SEQ, 8, _NW, 8, _LANES), jnp.float32),
        scratch_types=[
            pltpu.VMEM((_SGRP, 8, _LANES), jnp.int32),
            pltpu.VMEM((_SEQ, _D), jnp.float32),
            pltpu.VMEM((4, _LANES, _D), jnp.float32),
            pltpu.VMEM((_D, _LANES + 1), jnp.float32),
            pltpu.VMEM((_D, _LANES + 1), jnp.float32),
            pltpu.SemaphoreType.DMA,
            pltpu.SemaphoreType.DMA,
            pltpu.SemaphoreType.DMA,
            pltpu.SemaphoreType.DMA,
            pltpu.SemaphoreType.DMA,
            pltpu.SemaphoreType.DMA,
        ],
    )
    def k(table_hbm, xt_hbm, pe_hbm, out_hbm, idx_v, pe_v, gbuf, tbuf0,
          tbuf1, gsem0, gsem1, gsem2, gsem3, osem0, osem1):
        tbufs = (tbuf0, tbuf1)
        gsems = (gsem0, gsem1, gsem2, gsem3)
        osems = (osem0, osem1)
        wid = lax.axis_index("s") * _NC + lax.axis_index("c")
        pltpu.sync_copy(xt_hbm.at[:, wid], idx_v)
        pltpu.sync_copy(pe_hbm, pe_v)

        iota = lax.iota(jnp.int32, 16)
        d_vecs = [dg * 16 + iota for dg in range(4)]

        def gather_copies(s, b):
            cps = []
            for j in range(8):
                idxv = idx_v[s // 8, s % 8, pl.ds(j * 16, 16)]
                cps.append(pltpu.make_async_copy(
                    table_hbm.at[idxv], gbuf.at[b, pl.ds(j * 16, 16)],
                    gsems[b]))
            return cps

        def out_copy(s, tb, r):
            return pltpu.make_async_copy(
                tbufs[tb].at[pl.ds(r * 8, 8), pl.ds(0, _LANES)],
                out_hbm.at[s, r, wid], osems[tb])

        for cp in gather_copies(0, 0):
            cp.start()
        for cp in gather_copies(1, 1):
            cp.start()

        def body(g, carry):
            for b in range(4):
                s = 4 * g + b
                tb = b % 2

                @pl.when(s < _SEQ - 2)
                def _():
                    for cp in gather_copies(s + 2, (b + 2) % 4):
                        cp.start()

                for cp in gather_copies(s, b):
                    cp.wait()

                @pl.when(s >= 2)
                def _():
                    for r in range(8):
                        out_copy(s - 2, tb, r).wait()

                pe_vregs = [pe_v[s, pl.ds(dg * 16, 16)] for dg in range(4)]
                tbuf2d = tbufs[tb]

                @plsc.parallel_loop(0, _LANES, unroll=4)
                def _(e):
                    esplat = jnp.full((16,), e, jnp.int32)
                    for dg in range(4):
                        v = gbuf[b, e, pl.ds(dg * 16, 16)] + pe_vregs[dg]
                        plsc.store_scatter(tbuf2d, [d_vecs[dg], esplat], v)

                for r in range(8):
                    out_copy(s, tb, r).start()
            return carry

        lax.fori_loop(0, _SEQ // 4, body, 0)
        for r in range(8):
            out_copy(_SEQ - 2, 0, r).wait()
        for r in range(8):
            out_copy(_SEQ - 1, 1, r).wait()

    return k(table, xt, pe)


def kernel(x, table):
    xt = (x.astype(jnp.int32).T
          .reshape(_SGRP, 8, _NW, _LANES).transpose(0, 2, 1, 3))
    pe = _pos_encoding(_SEQ, _D)
    out5d = _sc_embed(table, xt, pe)
    out = out5d.transpose(2, 4, 0, 1, 3).reshape(_BATCH, _SEQ, _D)
    return out
